# baseline (speedup 1.0000x reference)
import functools

import jax
import jax.numpy as jnp
from jax.experimental import pallas as pl
from jax.experimental.pallas import tpu as pltpu

IN_F = 1024
OUT_F = 1024
S_T = 512
NCH = 4
CHUNK = IN_F // NCH
K = 2
MAX_SLOTS = 8


def _vm_kernel(meta_ref, p_ref, x_ref, whbm_ref, b0_ref, b1_ref, out_ref,
               w_cache, sems):
    b = pl.program_id(0)
    s = pl.program_id(1)
    nb = pl.num_programs(0)

    def _copy(bb, k, c):
        row = meta_ref[bb, k, 0]
        cslot = meta_ref[bb, k, 1]
        return pltpu.make_async_copy(
            whbm_ref.at[row, pl.ds(c * CHUNK, CHUNK), :],
            w_cache.at[cslot, pl.ds(c * CHUNK, CHUNK), :],
            sems.at[cslot],
        )

    @pl.when((b == 0) & (s == 0))
    def _first():
        for k in range(K):
            @pl.when(meta_ref[0, k, 2] == 1)
            def _go():
                for c in range(NCH):
                    _copy(0, k, c).start()

    @pl.when(b + 1 < nb)
    def _next():
        for k in range(K):
            @pl.when(meta_ref[b + 1, k, 2] == 1)
            def _go():
                _copy(b + 1, k, s).start()

    p0 = p_ref[b, 0]
    p1 = p_ref[b, 1]
    bias = p0 * b0_ref[0, 0] + p1 * b1_ref[0, 0]
    slot0 = meta_ref[b, 0, 1]
    slot1 = meta_ref[b, 1, 1]

    @pl.when(s == 0)
    def _first_step():
        acc = jnp.zeros_like(out_ref[0])
        for c in range(NCH):
            for k in range(K):
                @pl.when(meta_ref[b, k, 2] == 1)
                def _wait():
                    _copy(b, k, c).wait()
            sl = pl.ds(c * CHUNK, CHUNK)
            wc = p0 * w_cache[slot0, sl, :] + p1 * w_cache[slot1, sl, :]
            acc += jnp.dot(x_ref[0, :, sl], wc,
                           preferred_element_type=jnp.float32)
        out_ref[0] = acc + bias[None, :]

    @pl.when(s != 0)
    def _steady_step():
        w = p0 * w_cache[slot0] + p1 * w_cache[slot1]
        acc = jnp.dot(x_ref[0], w, preferred_element_type=jnp.float32)
        out_ref[0] = acc + bias[None, :]


@jax.jit
def kernel(x, selection_index, selection_probabilities, W_bank, b_bank):
    B, S, _ = x.shape
    grid = (B, S // S_T)
    b_bank3 = b_bank[:, None, :]

    flat = selection_index.reshape(-1).astype(jnp.int32)
    eq = flat[:, None] == flat[None, :]
    first_j = jnp.argmax(eq, axis=1)
    is_first = first_j == jnp.arange(flat.shape[0])
    slot_of_pos = jnp.cumsum(is_first.astype(jnp.int32)) - 1
    slot = slot_of_pos[first_j]
    meta = jnp.stack(
        [flat, slot, is_first.astype(jnp.int32)], axis=-1
    ).reshape(B, K, 3)

    grid_spec = pltpu.PrefetchScalarGridSpec(
        num_scalar_prefetch=1,
        grid=grid,
        in_specs=[
            pl.BlockSpec(memory_space=pltpu.SMEM),
            pl.BlockSpec((1, S_T, IN_F), lambda b, s, idx: (b, s, 0)),
            pl.BlockSpec(memory_space=pl.ANY),
            pl.BlockSpec((1, 1, OUT_F), lambda b, s, idx: (idx[b, 0, 0], 0, 0)),
            pl.BlockSpec((1, 1, OUT_F), lambda b, s, idx: (idx[b, 1, 0], 0, 0)),
        ],
        out_specs=pl.BlockSpec((1, S_T, OUT_F), lambda b, s, idx: (b, s, 0)),
        scratch_shapes=[
            pltpu.VMEM((MAX_SLOTS, IN_F, OUT_F), jnp.float32),
            pltpu.SemaphoreType.DMA((MAX_SLOTS,)),
        ],
    )

    out = pl.pallas_call(
        _vm_kernel,
        grid_spec=grid_spec,
        out_shape=jax.ShapeDtypeStruct((B, S, OUT_F), jnp.float32),
        compiler_params=pltpu.CompilerParams(
            dimension_semantics=("arbitrary", "arbitrary"),
        ),
    )(meta, selection_probabilities, x, W_bank, b_bank3, b_bank3)
    return out


# speedup vs baseline: 1.0715x; 1.0715x over previous
import functools

import jax
import jax.numpy as jnp
from jax.experimental import pallas as pl
from jax.experimental.pallas import tpu as pltpu

IN_F = 1024
OUT_F = 1024
S_T = 512
K = 2


def _vm_kernel(idx_ref, p_ref, x_ref, whbm_ref, b0_ref, b1_ref, out_ref,
               w_buf, sems):
    b = pl.program_id(0)
    s = pl.program_id(1)
    nb = pl.num_programs(0)
    ns = pl.num_programs(1)
    slot = jax.lax.rem(b, 2)
    chunk = IN_F // 4

    def _copy(src_row, dst_slot, c, k):
        return pltpu.make_async_copy(
            whbm_ref.at[src_row, pl.ds(c * chunk, chunk), :],
            w_buf.at[dst_slot, k, pl.ds(c * chunk, chunk), :],
            sems.at[dst_slot, k],
        )

    @pl.when((b == 0) & (s == 0))
    def _first():
        for c in range(4):
            for k in range(K):
                _copy(idx_ref[0, k], 0, c, k).start()

    @pl.when(b + 1 < nb)
    def _next():
        nslot = jax.lax.rem(b + 1, 2)
        for k in range(K):
            _copy(idx_ref[b + 1, k], nslot, s, k).start()

    p0 = p_ref[b, 0]
    p1 = p_ref[b, 1]
    bias = p0 * b0_ref[0, 0] + p1 * b1_ref[0, 0]

    @pl.when(s == 0)
    def _first_step():
        acc = jnp.zeros_like(out_ref[0])
        for c in range(4):
            for k in range(K):
                _copy(idx_ref[b, k], slot, c, k).wait()
            sl = pl.ds(c * chunk, chunk)
            wc = p0 * w_buf[slot, 0, sl, :] + p1 * w_buf[slot, 1, sl, :]
            acc += jnp.dot(x_ref[0, :, sl], wc,
                           preferred_element_type=jnp.float32)
        out_ref[0] = acc + bias[None, :]

    @pl.when(s != 0)
    def _steady_step():
        w = p0 * w_buf[slot, 0] + p1 * w_buf[slot, 1]
        acc = jnp.dot(x_ref[0], w, preferred_element_type=jnp.float32)
        out_ref[0] = acc + bias[None, :]


@jax.jit
def kernel(x, selection_index, selection_probabilities, W_bank, b_bank):
    B, S, _ = x.shape
    grid = (B, S // S_T)
    b_bank3 = b_bank[:, None, :]

    grid_spec = pltpu.PrefetchScalarGridSpec(
        num_scalar_prefetch=1,
        grid=grid,
        in_specs=[
            pl.BlockSpec(memory_space=pltpu.SMEM),
            pl.BlockSpec((1, S_T, IN_F), lambda b, s, idx: (b, s, 0)),
            pl.BlockSpec(memory_space=pl.ANY),
            pl.BlockSpec((1, 1, OUT_F), lambda b, s, idx: (idx[b, 0], 0, 0)),
            pl.BlockSpec((1, 1, OUT_F), lambda b, s, idx: (idx[b, 1], 0, 0)),
        ],
        out_specs=pl.BlockSpec((1, S_T, OUT_F), lambda b, s, idx: (b, s, 0)),
        scratch_shapes=[
            pltpu.VMEM((2, K, IN_F, OUT_F), jnp.float32),
            pltpu.SemaphoreType.DMA((2, K)),
        ],
    )

    out = pl.pallas_call(
        _vm_kernel,
        grid_spec=grid_spec,
        out_shape=jax.ShapeDtypeStruct((B, S, OUT_F), jnp.float32),
        compiler_params=pltpu.CompilerParams(
            dimension_semantics=("arbitrary", "arbitrary"),
        ),
    )(selection_index, selection_probabilities, x, W_bank, b_bank3, b_bank3)
    return out
